# baseline (speedup 1.0000x reference)
import functools

import jax
import jax.numpy as jnp
from jax import lax
from jax.experimental import pallas as pl
from jax.experimental.pallas import tpu as pltpu
from jax.experimental.pallas import tpu_sc as plsc

N = 100000
E = 1600000
G = 512
EPS = 1e-5

NC = 2
NS = 16
NW = NC * NS

NPAD = 100352
EB = 2000
NBLK = E // EB
BPW = NBLK // NW

RSH = 12
R = 1 << RSH
C = (N + R - 1) // R
CP = 32
CAP = E + NW * C * 8
NOUT = C * R
GB = 128

STG = 64
FLS = 48

_SC = None


def _popcnt(m):
    return jnp.sum(m.astype(jnp.int32))


def _build_sc():
    mesh = plsc.VectorSubcoreMesh(
        core_axis_name="c", subcore_axis_name="s",
        num_cores=NC, num_subcores=NS)

    def _wid():
        return lax.axis_index("s") * NC + lax.axis_index("c")


    @functools.partial(
        pl.kernel,
        out_type=jax.ShapeDtypeStruct((NC, NPAD), jnp.float32),
        mesh=mesh,
        scratch_types=[
            pltpu.VMEM_SHARED((NPAD,), jnp.float32),
            pltpu.VMEM((EB,), jnp.int32),
            pltpu.VMEM((EB,), jnp.float32),
            pltpu.VMEM((NPAD // NS,), jnp.float32),
        ],
    )
    def _deg_sc(dst_hbm, out_hbm, acc_sh, idx_v, ones_v, slab_v):
        c = lax.axis_index("c")
        s = lax.axis_index("s")
        slab = NPAD // NS

        @pl.loop(0, slab // 16)
        def _zero(i):
            slab_v[pl.ds(i * 16, 16)] = jnp.zeros((16,), jnp.float32)

        @pl.loop(0, EB // 16)
        def _one(i):
            ones_v[pl.ds(i * 16, 16)] = jnp.ones((16,), jnp.float32)

        pltpu.sync_copy(slab_v, acc_sh.at[pl.ds(s * slab, slab)])
        plsc.subcore_barrier()

        w = _wid()

        @pl.loop(0, BPW)
        def _blk(i):
            b = i * NW + w
            pltpu.sync_copy(dst_hbm.at[pl.ds(b * EB, EB)], idx_v)
            pltpu.sync_copy(ones_v, acc_sh.at[idx_v], add=True)

        plsc.subcore_barrier()
        pltpu.sync_copy(acc_sh.at[pl.ds(s * slab, slab)],
                        out_hbm.at[c, pl.ds(s * slab, slab)])


    @functools.partial(
        pl.kernel,
        out_type=jax.ShapeDtypeStruct((NW, CP), jnp.int32),
        mesh=mesh,
        scratch_types=[
            pltpu.VMEM((EB,), jnp.int32),
            pltpu.VMEM((CP,), jnp.int32),
            pltpu.SMEM((CP,), jnp.int32),
        ],
    )
    def _hist_sc(dst_hbm, hist_hbm, blk_v, row_v, cnt_s):
        w = _wid()

        for i in range(C):
            cnt_s[i] = 0

        @pl.loop(0, BPW)
        def _blk(i):
            b = i * NW + w
            pltpu.sync_copy(dst_hbm.at[pl.ds(b * EB, EB)], blk_v)

            @pl.loop(0, EB // 16)
            def _v(vj):
                dv = blk_v[pl.ds(vj * 16, 16)]
                bkv = jax.lax.shift_right_logical(dv, RSH)
                for bkt in range(C):
                    cnt_s[bkt] = cnt_s[bkt] + _popcnt(bkv == bkt)

        iota = lax.iota(jnp.int32, 16)
        for half in range(CP // 16):
            vec = jnp.zeros((16,), jnp.int32)
            for l in range(16):
                bkt = half * 16 + l
                if bkt < C:
                    vec = jnp.where(iota == l, cnt_s[bkt], vec)
            row_v[pl.ds(half * 16, 16)] = vec

        pltpu.sync_copy(row_v, hist_hbm.at[w])


    @functools.partial(
        pl.kernel,
        out_type=(
            jax.ShapeDtypeStruct((CAP,), jnp.int32),
            jax.ShapeDtypeStruct((CAP,), jnp.int32),
        ),
        mesh=mesh,
        scratch_types=[
            pltpu.VMEM((NW, CP), jnp.int32),
            pltpu.VMEM((EB,), jnp.int32),
            pltpu.VMEM((EB,), jnp.int32),
            pltpu.VMEM((C * STG,), jnp.int32),
            pltpu.VMEM((C * STG,), jnp.int32),
            pltpu.SMEM((CP,), jnp.int32),
            pltpu.SMEM((CP,), jnp.int32),
        ],
    )
    def _bucket_sc(src_hbm, dst_hbm, offs_hbm,
                   srcb_hbm, dstb_hbm,
                   offs_v, sblk_v, dblk_v, sstg_v, dstg_v, hcur_s, fill_s):
        w = _wid()
        pltpu.sync_copy(offs_hbm, offs_v)

        for half in range(CP // 16):
            sel = jnp.zeros((16,), jnp.int32)
            for w2 in range(NW):
                v = offs_v[w2, pl.ds(half * 16, 16)]
                sel = jnp.where(w2 == w, v, sel)
            for l in range(16):
                bkt = half * 16 + l
                if bkt < C:
                    hcur_s[bkt] = sel[l]

        for bkt in range(C):
            fill_s[bkt] = 0

        @pl.loop(0, BPW)
        def _blk(i):
            b = i * NW + w
            pltpu.sync_copy(src_hbm.at[pl.ds(b * EB, EB)], sblk_v)
            pltpu.sync_copy(dst_hbm.at[pl.ds(b * EB, EB)], dblk_v)

            @pl.loop(0, EB // 16)
            def _v(vj):
                sv = sblk_v[pl.ds(vj * 16, 16)]
                dv = dblk_v[pl.ds(vj * 16, 16)]
                bkv = jax.lax.shift_right_logical(dv, RSH)
                dloc = jnp.bitwise_and(dv, R - 1)
                for bkt in range(C):
                    m = bkv == bkt
                    cur = fill_s[bkt]
                    plsc.store_compressed(
                        sstg_v.at[pl.ds(bkt * STG + cur, 16)], sv, mask=m)
                    plsc.store_compressed(
                        dstg_v.at[pl.ds(bkt * STG + cur, 16)], dloc, mask=m)
                    cur2 = cur + _popcnt(m)

                    @pl.when(cur2 >= FLS)
                    def _flush():
                        hb = hcur_s[bkt]
                        pltpu.sync_copy(sstg_v.at[pl.ds(bkt * STG, FLS)],
                                        srcb_hbm.at[pl.ds(hb, FLS)])
                        pltpu.sync_copy(dstg_v.at[pl.ds(bkt * STG, FLS)],
                                        dstb_hbm.at[pl.ds(hb, FLS)])
                        rs = sstg_v[pl.ds(bkt * STG + FLS, 16)]
                        rd = dstg_v[pl.ds(bkt * STG + FLS, 16)]
                        sstg_v[pl.ds(bkt * STG, 16)] = rs
                        dstg_v[pl.ds(bkt * STG, 16)] = rd
                        hcur_s[bkt] = hb + FLS
                        fill_s[bkt] = cur2 - FLS

                    @pl.when(cur2 < FLS)
                    def _nofl():
                        fill_s[bkt] = cur2

        dummy_d = jnp.full((16,), R, jnp.int32)
        dummy_s = lax.iota(jnp.int32, 16)
        for bkt in range(C):
            f = fill_s[bkt]

            @pl.when(f > 0)
            def _tail():
                sstg_v[pl.ds(bkt * STG + f, 16)] = dummy_s
                dstg_v[pl.ds(bkt * STG + f, 16)] = dummy_d
                fp = jax.lax.shift_left(
                    jax.lax.shift_right_logical(f + 7, 3), 3)

                @pl.loop(0, jax.lax.shift_right_logical(fp, 3))
                def _fl(g):
                    cu = hcur_s[bkt] + g * 8
                    pltpu.sync_copy(
                        sstg_v.at[pl.ds(bkt * STG + g * 8, 8)],
                        srcb_hbm.at[pl.ds(cu, 8)])
                    pltpu.sync_copy(
                        dstg_v.at[pl.ds(bkt * STG + g * 8, 8)],
                        dstb_hbm.at[pl.ds(cu, 8)])


    def _make_agg(F):
        @functools.partial(
            pl.kernel,
            out_type=jax.ShapeDtypeStruct((NOUT, F), jnp.float32),
            mesh=mesh,
            compiler_params=pltpu.CompilerParams(use_tc_tiling_on_sc=False),
            scratch_types=[
                pltpu.VMEM_SHARED((R + 8, F), jnp.float32),
                pltpu.VMEM((R // NS, F), jnp.float32),
                pltpu.VMEM((GB,), jnp.int32),
                pltpu.VMEM((GB,), jnp.int32),
                pltpu.VMEM((GB, F), jnp.float32),
                pltpu.VMEM((8,), jnp.int32),
                pltpu.VMEM((8,), jnp.int32),
                pltpu.VMEM((8, F), jnp.float32),
                pltpu.VMEM((NW, CP), jnp.int32),
                pltpu.VMEM((NW, CP), jnp.int32),
                pltpu.SMEM((2 * CP,), jnp.int32),
                pltpu.SMEM((2 * CP,), jnp.int32),
                pltpu.SemaphoreType.DMA,
            ],
        )
        def _agg(t_hbm, srcb_hbm, dstb_hbm, offs_hbm, pcnt_hbm, out_hbm,
                 acc_sh, zslab_v, sidx_v, didx_v, gbuf_v,
                 tsidx_v, tdidx_v, tgbuf_v, offs_v, pcnt_v,
                 start_s, count_s, sem):
            c = lax.axis_index("c")
            s = lax.axis_index("s")
            rows = R // NS

            pltpu.sync_copy(offs_hbm, offs_v)
            pltpu.sync_copy(pcnt_hbm, pcnt_v)

            for r2 in range(2):
                for half in range(CP // 16):
                    so = jnp.zeros((16,), jnp.int32)
                    po = jnp.zeros((16,), jnp.int32)
                    for w2 in range(NW):
                        vo = offs_v[w2, pl.ds(half * 16, 16)]
                        vp = pcnt_v[w2, pl.ds(half * 16, 16)]
                        sel = w2 == (s * 2 + r2)
                        so = jnp.where(sel, vo, so)
                        po = jnp.where(sel, vp, po)
                    for l in range(16):
                        bkt = half * 16 + l
                        if bkt < C:
                            start_s[r2 * CP + bkt] = so[l]
                            count_s[r2 * CP + bkt] = po[l]

            @pl.loop(0, rows)
            def _z(i):
                for f in range(F // 16):
                    zslab_v[i, pl.ds(f * 16, 16)] = (
                        jnp.zeros((16,), jnp.float32))

            @pl.loop(c, C, step=NC)
            def _chunk(ci):
                pltpu.sync_copy(zslab_v, acc_sh.at[pl.ds(s * rows, rows)])
                plsc.subcore_barrier()

                for r2 in range(2):
                    start = pl.multiple_of(start_s[r2 * CP + ci], 8)
                    count = count_s[r2 * CP + ci]
                    nfull = jax.lax.shift_right_logical(count, 7)

                    @pl.loop(0, nfull)
                    def _b(k):
                        pos = pl.multiple_of(start + k * GB, 8)
                        pltpu.sync_copy(srcb_hbm.at[pl.ds(pos, GB)], sidx_v)
                        pltpu.sync_copy(dstb_hbm.at[pl.ds(pos, GB)], didx_v)
                        pltpu.async_copy(t_hbm.at[sidx_v], gbuf_v, sem).wait()
                        pltpu.sync_copy(gbuf_v, acc_sh.at[didx_v], add=True)

                    tail0 = start + nfull * GB
                    ntail = jax.lax.shift_right_logical(
                        count - nfull * GB, 3)

                    @pl.loop(0, ntail)
                    def _t(k):
                        pos = pl.multiple_of(tail0 + k * 8, 8)
                        pltpu.sync_copy(srcb_hbm.at[pl.ds(pos, 8)], tsidx_v)
                        pltpu.sync_copy(dstb_hbm.at[pl.ds(pos, 8)], tdidx_v)
                        pltpu.async_copy(
                            t_hbm.at[tsidx_v], tgbuf_v, sem).wait()
                        pltpu.sync_copy(tgbuf_v, acc_sh.at[tdidx_v], add=True)

                plsc.subcore_barrier()
                pltpu.sync_copy(
                    acc_sh.at[pl.ds(s * rows, rows)],
                    out_hbm.at[pl.ds(ci * R + s * rows, rows)])

        return _agg

    return {"deg": _deg_sc, "hist": _hist_sc, "bucket": _bucket_sc,
            "agg": {F: _make_agg(F) for F in (16, 32, 48, 64, 96)}}


def _sc():
    global _SC
    if _SC is None:
        _SC = _build_sc()
    return _SC


def _mm(a, b):
    return jax.lax.dot_general(
        a.astype(jnp.bfloat16), b.astype(jnp.bfloat16),
        (((1,), (0,)), ((), ())), preferred_element_type=jnp.float32)


def _bn(x, g, b):
    mu = jnp.mean(x, axis=0)
    var = jnp.mean((x - mu) ** 2, axis=0)
    return (x - mu) / jnp.sqrt(var + EPS) * g + b


def kernel(x, edge_index, batch, params):
    src, dst = edge_index[0], edge_index[1]
    sc = _sc()
    degp = sc['deg'](dst)
    deg = degp[0, :N] + degp[1, :N] + 1.0
    dinv = jax.lax.rsqrt(deg)

    bk = dst >> RSH
    order = jnp.argsort(bk, stable=True)
    cnts = jnp.bincount(bk, length=C)
    pc = (cnts + 7) // 8 * 8
    starts = jnp.concatenate(
        [jnp.zeros((1,), jnp.int32),
         jnp.cumsum(pc)[:-1].astype(jnp.int32)])
    sbk = bk[order]
    excl = jnp.concatenate(
        [jnp.zeros((1,), jnp.int32),
         jnp.cumsum(cnts)[:-1].astype(jnp.int32)])
    pos = starts[sbk] + jnp.arange(E, dtype=jnp.int32) - excl[sbk]
    srcb = jnp.zeros((CAP,), jnp.int32).at[pos].set(src[order])
    dstb = jnp.full((CAP,), R, jnp.int32).at[pos].set(
        (dst[order]) & (R - 1))
    offs = jnp.zeros((NW, CP), jnp.int32).at[0, :C].set(starts)
    pcnt = jnp.zeros((NW, CP), jnp.int32).at[0, :C].set(pc)

    h = x
    for i in range(1, 6):
        t = _mm(h, params['W%d' % i]) * dinv[:, None]
        s = sc['agg'][t.shape[1]](t, srcb, dstb, offs, pcnt)[:N]
        h = dinv[:, None] * (s + t) + params['b%d' % i]
        h = _bn(h, params['g%d' % i], params['be%d' % i])
        h = jax.nn.leaky_relu(h, 0.01)

    sums = jax.ops.segment_sum(h, batch, num_segments=G)
    cnt = jax.ops.segment_sum(jnp.ones((N,), jnp.float32), batch, num_segments=G)
    hg = sums / jnp.maximum(cnt, 1.0)[:, None]
    p = params
    h = _bn(_mm(hg, p['lW1']) + p['lb1'], p['lg1'], p['lbe1'])
    h = jax.nn.leaky_relu(h, 0.01)
    h = _bn(_mm(h, p['lW2']) + p['lb2'], p['lg2'], p['lbe2'])
    h = jax.nn.leaky_relu(h, 0.01)
    return _mm(h, p['lW3']) + p['lb3']
